# baseline (speedup 1.0000x reference)
import jax
import jax.numpy as jnp
from jax import lax
from jax.experimental import pallas as pl
from jax.experimental.pallas import tpu as pltpu
from jax.experimental.pallas import tpu_sc as plsc

SHIFT = 500000
B = 16384
D = 64
V = 1000000
L = 16
NC = 2
NS = 16
NW = NC * NS
BPW = B // NW
NCHUNK = 4
CHUNK = BPW // NCHUNK
TR = 8


def _rsqrt(x):
    i = lax.bitcast_convert_type(x, jnp.int32)
    i = jnp.int32(0x5F3759DF) - lax.shift_right_logical(i, 1)
    y = lax.bitcast_convert_type(i, jnp.float32)
    y = y * (1.5 - 0.5 * x * y * y)
    y = y * (1.5 - 0.5 * x * y * y)
    return y


def _body(offset_hbm, table_hbm, out_hbm, i0, i1, i2, i3, rows3_v, sem):
    idx_refs = [i0, i1, i2, i3]
    wid = lax.axis_index("s") * NC + lax.axis_index("c")
    base = wid * BPW
    table2 = table_hbm

    for j in range(NCHUNK):
        pltpu.sync_copy(offset_hbm.at[pl.ds(base + j * CHUNK, CHUNK)],
                        idx_refs[j])
    for j in range(NCHUNK):
        for i in range(CHUNK // L):
            sl = pl.ds(i * L, L)
            idx_refs[j][sl] = idx_refs[j][sl] + SHIFT

    descs = [
        pltpu.async_copy(table2.at[idx_refs[j]],
                         rows3_v.at[pl.ds(j * CHUNK, CHUNK)], sem)
        for j in range(NCHUNK)
    ]
    for d in descs:
        d.wait()

    @plsc.parallel_loop(0, BPW, unroll=2)
    def _row(r):
        sl = [(k >> 1, pl.ds((k & 1) * L, L)) for k in range(D // L)]
        c = [rows3_v[r, a, d] for a, d in sl]
        acc = c[0] * c[0]
        for k in range(1, D // L):
            acc = acc + c[k] * c[k]
        s = jnp.sum(acc)
        sv = jnp.broadcast_to(s, (L,))
        scale = jnp.minimum(1.0, 2.0 * _rsqrt(sv))
        for k, (a, d) in enumerate(sl):
            rows3_v[r, a, d] = c[k] * scale

    pltpu.sync_copy(rows3_v, out_hbm.at[pl.ds(base, BPW)])


@jax.jit
def kernel(offset, table):
    blocks = table.reshape(V, 2, D // 2)
    mesh = plsc.VectorSubcoreMesh(core_axis_name="c", subcore_axis_name="s",
                                  num_cores=NC, num_subcores=NS)
    run = pl.kernel(
        _body,
        out_type=jax.ShapeDtypeStruct((B, 2, D // 2), jnp.float32),
        mesh=mesh,
        scratch_types=[pltpu.VMEM((CHUNK,), jnp.int32)] * NCHUNK + [
            pltpu.VMEM((BPW, 2, D // 2), jnp.float32),
            pltpu.SemaphoreType.DMA,
        ],
        compiler_params=pltpu.CompilerParams(needs_layout_passes=False,
                                             use_tc_tiling_on_sc=False),
    )
    return run(offset, blocks).reshape(B, D)


# speedup vs baseline: 1.8040x; 1.8040x over previous
import jax
import jax.numpy as jnp
from jax import lax
from jax.experimental import pallas as pl
from jax.experimental.pallas import tpu as pltpu
from jax.experimental.pallas import tpu_sc as plsc

SHIFT = 500000
B = 16384
D = 64
V = 1000000
L = 16
NC = 2
NS = 16
NW = NC * NS
BPW = B // NW
NCHUNK = 4
CHUNK = BPW // NCHUNK


def _rsqrt(x):
    i = lax.bitcast_convert_type(x, jnp.int32)
    i = jnp.int32(0x5F3759DF) - lax.shift_right_logical(i, 1)
    y = lax.bitcast_convert_type(i, jnp.float32)
    y = y * (1.5 - 0.5 * x * y * y)
    y = y * (1.5 - 0.5 * x * y * y)
    return y


def _body(offset_hbm, table_hbm, out_hbm, i0, i1, i2, i3, rmod_v, rows3_v,
          stage, sem):
    idx_refs = [i0, i1, i2, i3]
    wid = lax.axis_index("s") * NC + lax.axis_index("c")
    base = wid * BPW

    for j in range(NCHUNK):
        pltpu.sync_copy(offset_hbm.at[pl.ds(base + j * CHUNK, CHUNK)],
                        idx_refs[j])
    for j in range(NCHUNK):
        for i in range(CHUNK // L):
            sl = pl.ds(i * L, L)
            v = idx_refs[j][sl] + SHIFT
            idx_refs[j][sl] = lax.shift_right_logical(v, 1)
            rmod_v[pl.ds(j * CHUNK + i * L, L)] = v & 1

    descs = [
        pltpu.async_copy(table_hbm.at[idx_refs[j]],
                         rows3_v.at[pl.ds(j * CHUNK, CHUNK)], sem)
        for j in range(NCHUNK)
    ]
    for d in descs:
        d.wait()

    @plsc.parallel_loop(0, BPW // L, unroll=1)
    def _grp(g):
        rv = rmod_v[pl.ds(g * L, L)]
        for j in range(L):
            r = g * L + j
            par = rv[j]
            c = [rows3_v[r, par, pl.ds(k * L, L)] for k in range(D // L)]
            acc = c[0] * c[0]
            for k in range(1, D // L):
                acc = acc + c[k] * c[k]
            s = jnp.sum(acc)
            sv = jnp.broadcast_to(s, (L,))
            scale = jnp.minimum(1.0, 2.0 * _rsqrt(sv))
            for k in range(D // L):
                stage[r, pl.ds(k * L, L)] = c[k] * scale

    pltpu.sync_copy(stage, out_hbm.at[pl.ds(base, BPW)])


@jax.jit
def kernel(offset, table):
    pairs = table.reshape(V // 2, 2, D)
    mesh = plsc.VectorSubcoreMesh(core_axis_name="c", subcore_axis_name="s",
                                  num_cores=NC, num_subcores=NS)
    run = pl.kernel(
        _body,
        out_type=jax.ShapeDtypeStruct((B, D), jnp.float32),
        mesh=mesh,
        scratch_types=[pltpu.VMEM((CHUNK,), jnp.int32)] * NCHUNK + [
            pltpu.VMEM((BPW,), jnp.int32),
            pltpu.VMEM((BPW, 2, D), jnp.float32),
            pltpu.VMEM((BPW, D), jnp.float32),
            pltpu.SemaphoreType.DMA,
        ],
        compiler_params=pltpu.CompilerParams(needs_layout_passes=False,
                                             use_tc_tiling_on_sc=False),
    )
    return run(offset, pairs)


# speedup vs baseline: 8.7492x; 4.8499x over previous
import jax
import jax.numpy as jnp
from jax import lax
from jax.experimental import pallas as pl
from jax.experimental.pallas import tpu as pltpu
from jax.experimental.pallas import tpu_sc as plsc

SHIFT = 500000
B = 16384
D = 64
V = 1000000
L = 16
NC = 2
NS = 16
NW = NC * NS
BPW = B // NW
G = 16
NCH = BPW // G
TR = 8


def _rsqrt(x):
    i = lax.bitcast_convert_type(x, jnp.int32)
    i = jnp.int32(0x5F3759DF) - lax.shift_right_logical(i, 1)
    y = lax.bitcast_convert_type(i, jnp.float32)
    y = y * (1.5 - 0.5 * x * y * y)
    y = y * (1.5 - 0.5 * x * y * y)
    return y


def _body(offset_hbm, table_hbm, out_hbm, off_v, tidx_v, rmod_v, g0, g1,
          stage, sem0, sem1):
    wid = lax.axis_index("s") * NC + lax.axis_index("c")
    base = wid * BPW

    pltpu.sync_copy(offset_hbm.at[pl.ds(base, BPW)], off_v)
    for i in range(BPW // L):
        v = off_v[pl.ds(i * L, L)] + SHIFT
        tidx_v[pl.ds(i * L, L)] = lax.shift_right_logical(v, 3)
        rmod_v[pl.ds(i * L, L)] = v & 7

    def fire(c, gbuf, sem):
        for g in range(G // L):
            tv = tidx_v[pl.ds(c * G + g * L, L)]
            for j in range(L):
                pltpu.async_copy(table_hbm.at[tv[j]], gbuf.at[g * L + j],
                                 sem)

    def drain(gbuf, sem):
        pltpu.make_async_copy(table_hbm.at[pl.ds(0, G)], gbuf, sem).wait()

    def process(c, gbuf):
        for g in range(G // L):
            rv = rmod_v[pl.ds(c * G + g * L, L)]
            for j in range(L):
                jj = g * L + j
                row = c * G + jj
                r = rv[j]
                ck = [gbuf[jj, r, pl.ds(k * L, L)] for k in range(D // L)]
                acc = ck[0] * ck[0]
                for k in range(1, D // L):
                    acc = acc + ck[k] * ck[k]
                s = jnp.sum(acc)
                sv = jnp.broadcast_to(s, (L,))
                scale = jnp.minimum(1.0, 2.0 * _rsqrt(sv))
                for k in range(D // L):
                    stage[row, pl.ds(k * L, L)] = ck[k] * scale

    fire(0, g0, sem0)
    fire(1, g1, sem1)

    @pl.loop(0, NCH, step=2)
    def _pair(c):
        drain(g0, sem0)
        process(c, g0)

        @pl.when(c + 2 < NCH)
        def _():
            fire(c + 2, g0, sem0)

        drain(g1, sem1)
        process(c + 1, g1)

        @pl.when(c + 3 < NCH)
        def _():
            fire(c + 3, g1, sem1)

    pltpu.sync_copy(stage, out_hbm.at[pl.ds(base, BPW)])


@jax.jit
def kernel(offset, table):
    blocks = table.reshape(V // TR, TR, D)
    mesh = plsc.VectorSubcoreMesh(core_axis_name="c", subcore_axis_name="s",
                                  num_cores=NC, num_subcores=NS)
    run = pl.kernel(
        _body,
        out_type=jax.ShapeDtypeStruct((B, D), jnp.float32),
        mesh=mesh,
        scratch_types=[
            pltpu.VMEM((BPW,), jnp.int32),
            pltpu.VMEM((BPW,), jnp.int32),
            pltpu.VMEM((BPW,), jnp.int32),
            pltpu.VMEM((G, TR, D), jnp.float32),
            pltpu.VMEM((G, TR, D), jnp.float32),
            pltpu.VMEM((BPW, D), jnp.float32),
            pltpu.SemaphoreType.DMA,
            pltpu.SemaphoreType.DMA,
        ],
        compiler_params=pltpu.CompilerParams(needs_layout_passes=False,
                                             use_tc_tiling_on_sc=True),
    )
    return run(offset, blocks)
